# Initial kernel scaffold; baseline (speedup 1.0000x reference)
import jax
import jax.numpy as jnp
from jax.experimental import pallas as pl


def kernel(node_strings, node_key, edge_index, edge_type, embedding, key_weight, value_weight, query, node_weight, target_weight):
    raise NotImplementedError("write your pallas kernel here")


# speedup vs baseline: 22.8141x; 22.8141x over previous
import functools
import math

import jax
import jax.numpy as jnp
from jax import lax
from jax.experimental import pallas as pl
from jax.experimental.pallas import tpu as pltpu
from jax.experimental.pallas import tpu_sc as plsc

N = 10000
E = 320000
D = 128
H = 8
DH = 16
NK = 32
NE = 16
L = 2
CPB = 2
NP = 10240
NW = 32
BLK = 80


def _mesh():
    return plsc.VectorSubcoreMesh(core_axis_name="c", subcore_axis_name="s")


def _sc_gather(table, idx, width):
    b = idx.shape[0]
    bpw = b // NW
    nchunks = bpw // BLK

    @functools.partial(
        pl.kernel,
        mesh=_mesh(),
        out_type=jax.ShapeDtypeStruct((b, width), jnp.float32),
        scratch_types=[
            pltpu.VMEM((BLK,), jnp.int32),
            pltpu.VMEM((BLK, width), jnp.float32),
            pltpu.SemaphoreType.DMA,
        ],
    )
    def k(table_hbm, idx_hbm, out_hbm, idx_v, rows_v, sem):
        wid = lax.axis_index("s") * 2 + lax.axis_index("c")
        base = wid * bpw

        def body(j, carry):
            off = base + j * BLK
            pltpu.sync_copy(idx_hbm.at[pl.ds(off, BLK)], idx_v)
            pltpu.async_copy(table_hbm.at[idx_v], rows_v, sem).wait()
            pltpu.sync_copy(rows_v, out_hbm.at[pl.ds(off, BLK)])
            return carry

        lax.fori_loop(0, nchunks, body, 0)

    return k(table, idx)


def _sc_scatter_add(vals, idx, nrows, width):
    b = idx.shape[0]
    bpw = b // NW
    nchunks = bpw // BLK
    stripe = nrows // 16

    @functools.partial(
        pl.kernel,
        mesh=_mesh(),
        out_type=jax.ShapeDtypeStruct((2, nrows, width), jnp.float32),
        scratch_types=[
            pltpu.VMEM((BLK,), jnp.int32),
            pltpu.VMEM((BLK, width), jnp.float32),
            pltpu.VMEM_SHARED((nrows, width), jnp.float32),
        ],
    )
    def k(vals_hbm, idx_hbm, out_hbm, idx_v, vals_v, acc_sh):
        cid = lax.axis_index("c")
        sid = lax.axis_index("s")
        base = (cid * 16 + sid) * bpw

        def zrow(i, carry):
            def zlane(l, c2):
                vals_v[i, pl.ds(l * 16, 16)] = jnp.zeros((16,), jnp.float32)
                return c2
            return lax.fori_loop(0, width // 16, zlane, carry)

        lax.fori_loop(0, BLK, zrow, 0)

        def zcopy(m, carry):
            pltpu.sync_copy(vals_v, acc_sh.at[pl.ds(sid * stripe + m * BLK, BLK)])
            return carry

        lax.fori_loop(0, stripe // BLK, zcopy, 0)
        plsc.subcore_barrier()

        def body(j, carry):
            off = base + j * BLK
            pltpu.sync_copy(idx_hbm.at[pl.ds(off, BLK)], idx_v)
            pltpu.sync_copy(vals_hbm.at[pl.ds(off, BLK)], vals_v)
            pltpu.sync_copy(vals_v, acc_sh.at[idx_v], add=True)
            return carry

        lax.fori_loop(0, nchunks, body, 0)
        plsc.subcore_barrier()
        pltpu.sync_copy(
            acc_sh.at[pl.ds(sid * stripe, stripe)],
            out_hbm.at[cid, pl.ds(sid * stripe, stripe)],
        )

    return k(vals, idx)


def _tc_proj(wstack, feat):
    tb = 256
    grid = (64, NP // tb)

    def body(w_ref, f_ref, o_ref):
        o_ref[0] = lax.dot_general(
            f_ref[...], w_ref[0], (((1,), (1,)), ((), ())),
            preferred_element_type=jnp.float32)

    return pl.pallas_call(
        body,
        grid=grid,
        in_specs=[
            pl.BlockSpec((1, D, D), lambda t, b: (t, 0, 0)),
            pl.BlockSpec((tb, D), lambda t, b: (b, 0)),
        ],
        out_specs=pl.BlockSpec((1, tb, D), lambda t, b: (t, b, 0)),
        out_shape=jax.ShapeDtypeStruct((64, NP, D), jnp.float32),
    )(wstack, feat)


def _tc_attn(ke, qe, ve, s_sel, s_bcast):
    tb = 512
    grid = (E // tb,)

    def body(k_ref, q_ref, v_ref, sel_ref, bc_ref, av_ref, ex_ref):
        p = k_ref[...] * q_ref[...]
        lg = lax.dot_general(p, sel_ref[...], (((1,), (0,)), ((), ())),
                             preferred_element_type=jnp.float32) * 0.25
        ex = jnp.exp(lg)
        exb = lax.dot_general(ex, bc_ref[...], (((1,), (0,)), ((), ())),
                              preferred_element_type=jnp.float32)
        av_ref[...] = v_ref[...] * exb
        ex_ref[...] = exb

    return pl.pallas_call(
        body,
        grid=grid,
        in_specs=[
            pl.BlockSpec((tb, D), lambda b: (b, 0)),
            pl.BlockSpec((tb, D), lambda b: (b, 0)),
            pl.BlockSpec((tb, D), lambda b: (b, 0)),
            pl.BlockSpec((D, H), lambda b: (0, 0)),
            pl.BlockSpec((H, D), lambda b: (0, 0)),
        ],
        out_specs=[
            pl.BlockSpec((tb, D), lambda b: (b, 0)),
            pl.BlockSpec((tb, D), lambda b: (b, 0)),
        ],
        out_shape=[
            jax.ShapeDtypeStruct((E, D), jnp.float32),
            jax.ShapeDtypeStruct((E, D), jnp.float32),
        ],
    )(ke, qe, ve, s_sel, s_bcast)


def _tc_finish(agg2, s2, nproj, feat_res, add_res):
    tb = 256
    grid = (NP // tb,)

    def body(a_ref, s_ref, n_ref, r_ref, o_ref):
        agg = a_ref[0] + a_ref[1]
        sb = s_ref[0] + s_ref[1]
        x = n_ref[...] + agg / (sb + 1e-9)
        x = jnp.maximum(x, 0.0)
        mu = jnp.mean(x, axis=1, keepdims=True)
        var = jnp.mean((x - mu) * (x - mu), axis=1, keepdims=True)
        x = (x - mu) * lax.rsqrt(var + 1e-5)
        if add_res:
            x = x + r_ref[...]
        o_ref[...] = x

    return pl.pallas_call(
        body,
        grid=grid,
        in_specs=[
            pl.BlockSpec((2, tb, D), lambda b: (0, b, 0)),
            pl.BlockSpec((2, tb, D), lambda b: (0, b, 0)),
            pl.BlockSpec((tb, D), lambda b: (b, 0)),
            pl.BlockSpec((tb, D), lambda b: (b, 0)),
        ],
        out_specs=pl.BlockSpec((tb, D), lambda b: (b, 0)),
        out_shape=jax.ShapeDtypeStruct((NP, D), jnp.float32),
    )(agg2, s2, nproj, feat_res)


def _tc_readout(twn, feat):
    tb = 512
    grid = (NP // tb,)

    def body(t_ref, f_ref, o_ref):
        @pl.when(pl.program_id(0) == 0)
        def _init():
            o_ref[...] = jnp.zeros((8, D), jnp.float32)

        o_ref[...] += jnp.sum(t_ref[...] * f_ref[...])

    return pl.pallas_call(
        body,
        grid=grid,
        in_specs=[
            pl.BlockSpec((tb, D), lambda b: (b, 0)),
            pl.BlockSpec((tb, D), lambda b: (b, 0)),
        ],
        out_specs=pl.BlockSpec((8, D), lambda b: (0, 0)),
        out_shape=jax.ShapeDtypeStruct((8, D), jnp.float32),
    )(twn, feat)


def kernel(node_strings, node_key, edge_index, edge_type, embedding,
           key_weight, value_weight, query, node_weight, target_weight):
    src = edge_index[0].astype(jnp.int32)
    dst = edge_index[1].astype(jnp.int32)
    et = edge_type.astype(jnp.int32)
    nk = node_key.astype(jnp.int32)

    pad = NP - N
    keyp = jnp.concatenate([nk, jnp.full((pad,), NK, jnp.int32)])
    emb_idx = jnp.concatenate([node_strings.astype(jnp.int32),
                               jnp.zeros((pad,), jnp.int32)])
    idx_k = et * NP + src
    idx_v = (16 + et) * NP + src
    rows = jnp.arange(NP, dtype=jnp.int32)
    idx_n = jnp.where(keyp < NK, (32 + keyp) * NP + rows, 0)

    dd = jnp.arange(D) // DH
    s_sel = (dd[:, None] == jnp.arange(H)[None, :]).astype(jnp.float32)
    s_bcast = s_sel.T

    feat = _sc_gather(embedding, emb_idx, D)

    partials = []
    for li in range(L):
        wstack = jnp.concatenate([
            key_weight[li].reshape(NE, D, D),
            value_weight[li].reshape(NE, D, D),
            node_weight[li].reshape(NK, D, D),
        ], axis=0)
        q33 = jnp.concatenate([query[li].reshape(NK, D),
                               jnp.zeros((1, D), jnp.float32)])
        tw33 = jnp.concatenate([target_weight[li].reshape(NK, D),
                                jnp.zeros((1, D), jnp.float32)])
        q_slot = _sc_gather(q33, keyp, D)
        feat_in = feat
        for ci in range(CPB):
            pall = _tc_proj(wstack, feat).reshape(64 * NP, D)
            ke = _sc_gather(pall, idx_k, D)
            ve = _sc_gather(pall, idx_v, D)
            qe = _sc_gather(q_slot, dst, D)
            nproj = _sc_gather(pall, idx_n, D)
            attn_v, exb = _tc_attn(ke, qe, ve, s_sel, s_bcast)
            agg2 = _sc_scatter_add(attn_v, dst, NP, D)
            s2 = _sc_scatter_add(exb, dst, NP, D)
            feat = _tc_finish(agg2, s2, nproj, feat_in,
                              add_res=(ci == CPB - 1))
        twn = _sc_gather(tw33, keyp, D)
        partials.append(_tc_readout(twn, feat)[0, 0] / N)

    return ((partials[0] + partials[1]) * 0.5).reshape(1)
